# baseline (speedup 1.0000x reference)
import functools

import jax
import jax.numpy as jnp
from jax import lax
from jax.experimental import pallas as pl
from jax.experimental.pallas import tpu as pltpu
from jax.experimental.pallas import tpu_sc as plsc

NC = 2
NS = 16
NW = NC * NS
L = 16

B = 16384
H = 50
D = 16
RPW = B // NW
RING = 8
CH = 4
NCHUNK = RPW // CH

_mesh = plsc.VectorSubcoreMesh(
    core_axis_name="c", subcore_axis_name="s", num_cores=NC, num_subcores=NS
)


def _pool(buf_ref, b, t, lo):
    accs = [buf_ref[b, t, lo + j, :] for j in range(4)]
    for j in range(4, H):
        accs[j % 4] = accs[j % 4] + buf_ref[b, t, lo + j, :]
    return (accs[0] + accs[1]) + (accs[2] + accs[3])


@functools.partial(
    pl.kernel,
    out_type=jax.ShapeDtypeStruct((NW, L), jnp.float32),
    mesh=_mesh,
    scratch_types=[
        pltpu.VMEM((RPW * H,), jnp.int32),
        pltpu.VMEM((RPW * H,), jnp.int32),
        pltpu.VMEM((RPW,), jnp.float32),
        pltpu.VMEM((RING, 2, CH * H, L), jnp.float32),
        pltpu.VMEM((L,), jnp.float32),
    ] + [pltpu.SemaphoreType.DMA] * RING,
    compiler_params=pltpu.CompilerParams(
        use_tc_tiling_on_sc=False, needs_layout_passes=False),
)
def _team_score_kernel(idx1_hbm, idx2_hbm, res_hbm, table_hbm, out_hbm,
                       idx1_v, idx2_v, res_v, buf_v, out_v, *sems):
    wid = lax.axis_index("s") * NC + lax.axis_index("c")

    pltpu.sync_copy(idx1_hbm.at[pl.ds(wid * (RPW * H), RPW * H)], idx1_v)
    pltpu.sync_copy(idx2_hbm.at[pl.ds(wid * (RPW * H), RPW * H)], idx2_v)
    pltpu.sync_copy(res_hbm.at[pl.ds(wid * RPW, RPW)], res_v)

    def fire(c, b):
        pltpu.async_copy(
            table_hbm.at[idx1_v.at[pl.ds(c * (CH * H), CH * H)]],
            buf_v.at[b, 0], sems[b])
        pltpu.async_copy(
            table_hbm.at[idx2_v.at[pl.ds(c * (CH * H), CH * H)]],
            buf_v.at[b, 1], sems[b])

    def wait(c, b):
        pltpu.make_async_copy(
            table_hbm.at[idx1_v.at[pl.ds(c * (CH * H), CH * H)]],
            buf_v.at[b, 0], sems[b]
        ).wait()
        pltpu.make_async_copy(
            table_hbm.at[idx2_v.at[pl.ds(c * (CH * H), CH * H)]],
            buf_v.at[b, 1], sems[b]
        ).wait()

    for b in range(RING):
        fire(b, b)

    def chunk(b, c, lb, resv, total):
        wait(c, b)
        for i in range(CH):
            s1 = _pool(buf_v, b, 0, i * H)
            s2 = _pool(buf_v, b, 1, i * H)
            sig = 1.0 / (1.0 + jnp.exp(s2 - s1))
            t = sig * 2.0 - 1.0
            lane = lax.iota(jnp.int32, L) == (lb + i)
            rs = jnp.full((L,), jnp.sum(jnp.where(lane, resv, 0.0)))
            total = total + jnp.abs(t - rs)
        fire(jnp.minimum(c + RING, NCHUNK - 1), b)
        return total

    def body(k, total):
        for g in range(RING * CH // 16):
            resv = res_v[pl.ds(k * (RING * CH) + g * L, L)]
            for bb in range(16 // CH):
                b = g * (16 // CH) + bb
                total = chunk(b, k * RING + b, bb * CH, resv, total)
        return total

    total = lax.fori_loop(
        0, NCHUNK // RING, body, jnp.zeros((L,), jnp.float32)
    )

    for b in range(RING):
        wait(NCHUNK - 1, b)

    out_v[...] = total
    pltpu.sync_copy(out_v, out_hbm.at[wid])


def kernel(team_1, team_2, result, emb_table):
    t1 = team_1.astype(jnp.int32).reshape(B * H)
    t2 = team_2.astype(jnp.int32).reshape(B * H)
    res = result.reshape(B)
    partials = _team_score_kernel(t1, t2, res, emb_table.astype(jnp.float32))
    return jnp.sum(partials) / jnp.float32(B * D)


# speedup vs baseline: 1.1292x; 1.1292x over previous
import functools

import jax
import jax.numpy as jnp
from jax import lax
from jax.experimental import pallas as pl
from jax.experimental.pallas import tpu as pltpu
from jax.experimental.pallas import tpu_sc as plsc

NC = 2
NS = 16
NW = NC * NS
L = 16

B = 16384
H = 50
D = 16
RPW = B // NW
RING = 4
CH = 4
NCHUNK = RPW // CH

_mesh = plsc.VectorSubcoreMesh(
    core_axis_name="c", subcore_axis_name="s", num_cores=NC, num_subcores=NS
)


def _pool(buf_ref, b, t, lo):
    accs = [buf_ref[b, t, lo + j, :] for j in range(4)]
    for j in range(4, H):
        accs[j % 4] = accs[j % 4] + buf_ref[b, t, lo + j, :]
    return (accs[0] + accs[1]) + (accs[2] + accs[3])


@functools.partial(
    pl.kernel,
    out_type=jax.ShapeDtypeStruct((NW, L), jnp.float32),
    mesh=_mesh,
    scratch_types=[
        pltpu.VMEM((RPW * H,), jnp.int32),
        pltpu.VMEM((RPW * H,), jnp.int32),
        pltpu.VMEM((RPW,), jnp.float32),
        pltpu.VMEM((RING, 2, CH * H, L), jnp.float32),
        pltpu.VMEM((L,), jnp.float32),
    ] + [pltpu.SemaphoreType.DMA] * RING,
    compiler_params=pltpu.CompilerParams(
        use_tc_tiling_on_sc=False, needs_layout_passes=False),
)
def _team_score_kernel(idx1_hbm, idx2_hbm, res_hbm, table_hbm, out_hbm,
                       idx1_v, idx2_v, res_v, buf_v, out_v, *sems):
    wid = lax.axis_index("s") * NC + lax.axis_index("c")

    pltpu.sync_copy(idx1_hbm.at[pl.ds(wid * (RPW * H), RPW * H)], idx1_v)
    pltpu.sync_copy(idx2_hbm.at[pl.ds(wid * (RPW * H), RPW * H)], idx2_v)
    pltpu.sync_copy(res_hbm.at[pl.ds(wid * RPW, RPW)], res_v)

    def fire(c, b):
        pltpu.async_copy(
            table_hbm.at[idx1_v.at[pl.ds(c * (CH * H), CH * H)]],
            buf_v.at[b, 0], sems[b])
        pltpu.async_copy(
            table_hbm.at[idx2_v.at[pl.ds(c * (CH * H), CH * H)]],
            buf_v.at[b, 1], sems[b])

    def wait(c, b):
        pltpu.make_async_copy(
            table_hbm.at[idx1_v.at[pl.ds(c * (CH * H), CH * H)]],
            buf_v.at[b, 0], sems[b]
        ).wait()
        pltpu.make_async_copy(
            table_hbm.at[idx2_v.at[pl.ds(c * (CH * H), CH * H)]],
            buf_v.at[b, 1], sems[b]
        ).wait()

    for b in range(RING):
        fire(b, b)

    def chunk(b, c, lb, resv, total):
        wait(c, b)
        for i in range(CH):
            s1 = _pool(buf_v, b, 0, i * H)
            s2 = _pool(buf_v, b, 1, i * H)
            sig = 1.0 / (1.0 + jnp.exp(s2 - s1))
            t = sig * 2.0 - 1.0
            lane = lax.iota(jnp.int32, L) == (lb + i)
            rs = jnp.full((L,), jnp.sum(jnp.where(lane, resv, 0.0)))
            total = total + jnp.abs(t - rs)
        fire(jnp.minimum(c + RING, NCHUNK - 1), b)
        return total

    def body(k, total):
        for g in range(RING * CH // 16):
            resv = res_v[pl.ds(k * (RING * CH) + g * L, L)]
            for bb in range(16 // CH):
                b = g * (16 // CH) + bb
                total = chunk(b, k * RING + b, bb * CH, resv, total)
        return total

    total = lax.fori_loop(
        0, NCHUNK // RING, body, jnp.zeros((L,), jnp.float32)
    )

    for b in range(RING):
        wait(NCHUNK - 1, b)

    out_v[...] = total
    pltpu.sync_copy(out_v, out_hbm.at[wid])


def kernel(team_1, team_2, result, emb_table):
    t1 = team_1.astype(jnp.int32).reshape(B * H)
    t2 = team_2.astype(jnp.int32).reshape(B * H)
    res = result.reshape(B)
    partials = _team_score_kernel(t1, t2, res, emb_table.astype(jnp.float32))
    return jnp.sum(partials) / jnp.float32(B * D)
